# baseline (speedup 1.0000x reference)
import jax
import jax.numpy as jnp
from jax.experimental import pallas as pl

N_NODES = 50000
N_REL = 4
D = 64
B = 16384
DEG_THRESH = 20
BM = 2048


def _combine_body(cat, num, des, post, dm, pm, deg, gr, gp,
                  wf, bf, wc, bc, expert_ref, prob_ref):
    fused = jnp.concatenate(
        [cat[...], num[...], des[...] * dm[...], post[...] * pm[...]], axis=1)
    fr = jnp.maximum(
        jnp.dot(fused, wf[...], preferred_element_type=jnp.float32) + bf[...], 0.0)
    fp = jax.nn.sigmoid(
        jnp.dot(fr, wc[...], preferred_element_type=jnp.float32) + bc[...])
    iso = deg[...] <= DEG_THRESH
    expert_ref[...] = jnp.where(iso, fr, gr[...])
    prob_ref[...] = jnp.where(iso, fp, gp[...])


def kernel(node_indices, degree, cat_repr, num_repr, des_repr, post_repr,
           des_mask, post_mask, edge_index, edge_type,
           W_fuse, b_fuse, W_cls_f, b_cls_f,
           node_emb, W_rel, W_self, b_graph, W_cls_g, b_cls_g):
    src = edge_index[0]
    dst = edge_index[1]
    h_all = jnp.einsum('nd,rde->rne', node_emb, W_rel)
    msg = h_all[edge_type, src]
    agg = jax.ops.segment_sum(msg, dst, num_segments=node_emb.shape[0])
    deg_in = jax.ops.segment_sum(jnp.ones_like(dst, dtype=jnp.float32), dst,
                                 num_segments=node_emb.shape[0])
    agg = agg / jnp.maximum(deg_in, 1.0)[:, None]
    h = jax.nn.relu(agg + node_emb @ W_self + b_graph)
    graph_repr = h[node_indices]
    graph_prob = jax.nn.sigmoid(graph_repr @ W_cls_g + b_cls_g)

    dm = des_mask.astype(jnp.float32)[:, None]
    pm = post_mask.astype(jnp.float32)[:, None]
    deg2 = degree.astype(jnp.int32)[:, None]

    row = lambda i: (i, 0)
    whole = lambda i: (0, 0)
    grid = B // BM
    expert, prob = pl.pallas_call(
        _combine_body,
        grid=(grid,),
        in_specs=[
            pl.BlockSpec((BM, D), row), pl.BlockSpec((BM, D), row),
            pl.BlockSpec((BM, D), row), pl.BlockSpec((BM, D), row),
            pl.BlockSpec((BM, 1), row), pl.BlockSpec((BM, 1), row),
            pl.BlockSpec((BM, 1), row),
            pl.BlockSpec((BM, D), row), pl.BlockSpec((BM, 1), row),
            pl.BlockSpec((4 * D, D), whole), pl.BlockSpec((1, D), whole),
            pl.BlockSpec((D, 1), whole), pl.BlockSpec((1, 1), whole),
        ],
        out_specs=[pl.BlockSpec((BM, D), row), pl.BlockSpec((BM, 1), row)],
        out_shape=[
            jax.ShapeDtypeStruct((B, D), jnp.float32),
            jax.ShapeDtypeStruct((B, 1), jnp.float32),
        ],
    )(cat_repr, num_repr, des_repr, post_repr, dm, pm, deg2,
      graph_repr, graph_prob,
      W_fuse, b_fuse.reshape(1, D), W_cls_f, b_cls_f.reshape(1, 1))
    return expert, prob


# speedup vs baseline: 9.6099x; 9.6099x over previous
import dataclasses
import functools

import jax
import jax.numpy as jnp
from jax import lax
from jax.experimental import pallas as pl
from jax.experimental.pallas import tpu as pltpu
from jax.experimental.pallas import tpu_sc as plsc

N = 50000
E = 800000
R = 4
D = 64
B = 16384
DEG_THRESH = 20

NTILES = 32
EP = 819200
EPT = EP // NTILES
CH1 = 3200
NCH1 = EPT // CH1
NCH2 = EPT // 512
BP = 16512
DUMMY = B
CR = 1152
BPT = B // NTILES
HREL = R * N
MARKN = 50048
RPT = BP // 16


def _sc_compiler_params():
    cp = pltpu.CompilerParams()
    fields = pltpu.CompilerParams.__dataclass_fields__
    if "needs_layout_passes" in fields:
        cp = dataclasses.replace(cp, needs_layout_passes=False)
    if "use_tc_tiling_on_sc" in fields:
        cp = dataclasses.replace(cp, use_tc_tiling_on_sc=False)
    return cp


def _sc_mesh():
    return plsc.VectorSubcoreMesh(core_axis_name="c", subcore_axis_name="s")


def _hall_body(emb, wrel, out):
    out[...] = jnp.dot(emb[...], wrel[0], preferred_element_type=jnp.float32)


def _hall(node_emb, W_rel):
    nb = 25
    bm = N // nb
    return pl.pallas_call(
        _hall_body,
        grid=(R, nb),
        in_specs=[pl.BlockSpec((bm, D), lambda r, i: (i, 0)),
                  pl.BlockSpec((1, D, D), lambda r, i: (r, 0, 0))],
        out_specs=pl.BlockSpec((bm, D), lambda r, i: (r * nb + i, 0)),
        out_shape=jax.ShapeDtypeStruct((HREL, D), jnp.float32),
    )(node_emb, W_rel)


def _route_pass(nidx, srce, dste, typee):
    out_type = [
        jax.ShapeDtypeStruct((EP,), jnp.int32),
        jax.ShapeDtypeStruct((EP,), jnp.int32),
        jax.ShapeDtypeStruct((2, CR, 16), jnp.float32),
        jax.ShapeDtypeStruct((B,), jnp.int32),
    ]
    scratch = [
        pltpu.VMEM((MARKN,), jnp.int32),
        pltpu.VMEM((2048,), jnp.int32),
        pltpu.VMEM((CH1,), jnp.int32),
        pltpu.VMEM((CH1,), jnp.int32),
        pltpu.VMEM((CH1,), jnp.int32),
        pltpu.VMEM((CH1,), jnp.int32),
        pltpu.VMEM((CH1,), jnp.int32),
        pltpu.VMEM((CR, 16), jnp.float32),
        pltpu.VMEM((72, 16), jnp.float32),
        pltpu.VMEM((9, 128), jnp.int32),
        pltpu.VMEM((512,), jnp.int32),
        pltpu.VMEM_SHARED((CR, 16), jnp.float32),
    ]

    @functools.partial(pl.kernel, mesh=_sc_mesh(), out_type=out_type,
                       scratch_types=scratch,
                       compiler_params=_sc_compiler_params())
    def body(nidx_h, src_h, dst_h, typ_h,
             slote_out, ride_out, cnt_out, slots_out,
             mark, nbuf, esrc, edst, etyp, sloto, rido, cnt_v,
             zcnt, ident, slots_v, cnt_sh):
        cid = lax.axis_index("c")
        sid = lax.axis_index("s")
        w = cid * 16 + sid
        i16 = lax.iota(jnp.int32, 16)
        zf16 = jnp.zeros((16,), jnp.float32)
        ones16 = jnp.ones((16,), jnp.float32)

        @pl.loop(0, 72)
        def _(rr):
            zcnt[rr, pl.ds(0, 16)] = zf16

        @pl.loop(0, CR)
        def _(rr):
            cnt_v[rr, pl.ds(0, 16)] = zf16

        @pl.loop(0, MARKN, step=16)
        def _(i):
            mark[pl.ds(i, 16)] = jnp.full((16,), -1, jnp.int32)

        for j in range(9):
            @pl.loop(0, 128, step=16)
            def _(o, j=j):
                ident[j, pl.ds(o, 16)] = (j * 128 + o) + i16

        pltpu.sync_copy(zcnt, cnt_sh.at[pl.ds(sid * 72, 72)])

        for bk in range(B // 2048):
            pltpu.sync_copy(nidx_h.at[pl.ds(bk * 2048, 2048)], nbuf)

            @pl.loop(0, 2048, step=16)
            def _(i, bk=bk):
                idx = nbuf[pl.ds(i, 16)]
                plsc.store_scatter(mark, [idx], bk * 2048 + i + i16)

        pltpu.sync_copy(nidx_h.at[pl.ds(w * BPT, BPT)], nbuf.at[pl.ds(0, BPT)])

        @pl.loop(0, BPT, step=16)
        def _(i):
            nv = nbuf[pl.ds(i, 16)]
            slots_v[pl.ds(i, 16)] = plsc.load_gather(mark, [nv])

        pltpu.sync_copy(slots_v, slots_out.at[pl.ds(w * BPT, BPT)])

        @pl.loop(0, NCH1)
        def _(c):
            base = w * EPT + c * CH1
            pltpu.sync_copy(src_h.at[pl.ds(base, CH1)], esrc)
            pltpu.sync_copy(dst_h.at[pl.ds(base, CH1)], edst)
            pltpu.sync_copy(typ_h.at[pl.ds(base, CH1)], etyp)

            @pl.loop(0, CH1, step=16)
            def _(o, base=base):
                d16 = edst[pl.ds(o, 16)]
                m = plsc.load_gather(mark, [d16])
                g = base + o + i16
                keep = (m >= 0) & (g < E)
                mm = jnp.where(keep, m, DUMMY)
                sloto[pl.ds(o, 16)] = mm
                rido[pl.ds(o, 16)] = etyp[pl.ds(o, 16)] * N + esrc[pl.ds(o, 16)]
                plsc.addupdate_scatter(
                    cnt_v,
                    [lax.shift_right_logical(mm, 4), lax.bitwise_and(mm, 15)],
                    ones16)

            pltpu.sync_copy(sloto, slote_out.at[pl.ds(base, CH1)])
            pltpu.sync_copy(rido, ride_out.at[pl.ds(base, CH1)])

        plsc.subcore_barrier()
        for j in range(9):
            pltpu.sync_copy(cnt_v.at[pl.ds(j * 128, 128)],
                            cnt_sh.at[ident.at[j]], add=True)
        plsc.subcore_barrier()
        pltpu.sync_copy(cnt_sh.at[pl.ds(sid * 72, 72)],
                        cnt_out.at[cid, pl.ds(sid * 72, 72)])

    return body(nidx, srce, dste, typee)


def _scatter_pass(hall, slote2d, ride2d):
    out_type = jax.ShapeDtypeStruct((2, BP, D), jnp.float32)
    scratch = [
        pltpu.VMEM((4, 128), jnp.int32),
        pltpu.VMEM((4, 128), jnp.int32),
        pltpu.VMEM((128, D), jnp.float32),
        pltpu.VMEM_SHARED((BP, D), jnp.float32),
    ]

    @functools.partial(pl.kernel, mesh=_sc_mesh(), out_type=out_type,
                       scratch_types=scratch,
                       compiler_params=_sc_compiler_params())
    def body(hall_h, slote_h, ride_h, acc_out, slot_b, rid_b, rows_v, acc_sh):
        cid = lax.axis_index("c")
        sid = lax.axis_index("s")
        w = cid * 16 + sid
        zf16 = jnp.zeros((16,), jnp.float32)

        @pl.loop(0, 128)
        def _(rr):
            for cc in range(4):
                rows_v[rr, pl.ds(cc * 16, 16)] = zf16

        for off in range(0, RPT - 8, 128):
            pltpu.sync_copy(rows_v, acc_sh.at[pl.ds(sid * RPT + off, 128)])
        pltpu.sync_copy(rows_v.at[pl.ds(0, 8)],
                        acc_sh.at[pl.ds(sid * RPT + RPT - 8, 8)])
        plsc.subcore_barrier()

        @pl.loop(0, NCH2)
        def _(c):
            rowbase = w * (EPT // 128) + c * 4
            pltpu.sync_copy(slote_h.at[pl.ds(rowbase, 4)], slot_b)
            pltpu.sync_copy(ride_h.at[pl.ds(rowbase, 4)], rid_b)
            for j in range(4):
                pltpu.sync_copy(hall_h.at[rid_b.at[j]], rows_v)
                pltpu.sync_copy(rows_v, acc_sh.at[slot_b.at[j]], add=True)

        plsc.subcore_barrier()
        pltpu.sync_copy(acc_sh.at[pl.ds(sid * RPT, RPT)],
                        acc_out.at[cid, pl.ds(sid * RPT, RPT)])

    return body(hall, slote2d, ride2d)


def _gather_pass(slots2d, nidx2d, acc0, acc1, cnt0, cnt1, node_emb):
    out_type = [
        jax.ShapeDtypeStruct((B, D), jnp.float32),
        jax.ShapeDtypeStruct((B,), jnp.float32),
        jax.ShapeDtypeStruct((B, D), jnp.float32),
    ]
    scratch = [
        pltpu.VMEM((4, 128), jnp.int32),
        pltpu.VMEM((4, 128), jnp.int32),
        pltpu.VMEM((256, D), jnp.float32),
        pltpu.VMEM((256, D), jnp.float32),
        pltpu.VMEM((256, D), jnp.float32),
        pltpu.VMEM((CR, 16), jnp.float32),
        pltpu.VMEM((CR, 16), jnp.float32),
        pltpu.VMEM((512,), jnp.float32),
    ]

    @functools.partial(pl.kernel, mesh=_sc_mesh(), out_type=out_type,
                       scratch_types=scratch,
                       compiler_params=_sc_compiler_params())
    def body(slots_h, nidx_h, acc0_h, acc1_h, cnt0_h, cnt1_h, emb_h,
             agg_out, cntb_out, embb_out,
             slots_v, nidx_v, rows0, rows1, erows, cnt0_v, cnt1_v, cntb_v):
        cid = lax.axis_index("c")
        sid = lax.axis_index("s")
        w = cid * 16 + sid
        pltpu.sync_copy(slots_h.at[pl.ds(w * 4, 4)], slots_v)
        pltpu.sync_copy(nidx_h.at[pl.ds(w * 4, 4)], nidx_v)
        pltpu.sync_copy(cnt0_h, cnt0_v)
        pltpu.sync_copy(cnt1_h, cnt1_v)
        for k in range(2):
            for j in range(2):
                rr = k * 2 + j
                pltpu.sync_copy(acc0_h.at[slots_v.at[rr]],
                                rows0.at[pl.ds(j * 128, 128)])
                pltpu.sync_copy(acc1_h.at[slots_v.at[rr]],
                                rows1.at[pl.ds(j * 128, 128)])
                pltpu.sync_copy(emb_h.at[nidx_v.at[rr]],
                                erows.at[pl.ds(j * 128, 128)])

            @pl.loop(0, 256)
            def _(rr):
                for cc in range(4):
                    sl = pl.ds(cc * 16, 16)
                    rows0[rr, sl] = rows0[rr, sl] + rows1[rr, sl]

            for j in range(2):
                rr = k * 2 + j

                @pl.loop(0, 128, step=16)
                def _(o, rr=rr, j=j, k=k):
                    s16 = slots_v[rr, pl.ds(o, 16)]
                    hi = lax.shift_right_logical(s16, 4)
                    lo = lax.bitwise_and(s16, 15)
                    c0 = plsc.load_gather(cnt0_v, [hi, lo])
                    c1 = plsc.load_gather(cnt1_v, [hi, lo])
                    cntb_v[pl.ds(k * 256 + j * 128 + o, 16)] = c0 + c1

            pltpu.sync_copy(rows0, agg_out.at[pl.ds(w * 512 + k * 256, 256)])
            pltpu.sync_copy(erows, embb_out.at[pl.ds(w * 512 + k * 256, 256)])
        pltpu.sync_copy(cntb_v, cntb_out.at[pl.ds(w * 512, 512)])

    return body(slots2d, nidx2d, acc0, acc1, cnt0, cnt1, node_emb)


BM = 2048


def _final_body(aggr, cntb, embb, deg, cat, num, des, post, dm, pm,
                wself, bgraph, wclsg, bclsg, wfuse, bfuse, wclsf, bclsf,
                expert_ref, prob_ref):
    agg = aggr[...] / jnp.maximum(cntb[...], 1.0)
    h = jnp.maximum(
        agg + jnp.dot(embb[...], wself[...],
                      preferred_element_type=jnp.float32) + bgraph[...], 0.0)
    gp = jax.nn.sigmoid(
        jnp.dot(h, wclsg[...], preferred_element_type=jnp.float32) + bclsg[...])
    fused = jnp.concatenate(
        [cat[...], num[...], des[...] * dm[...], post[...] * pm[...]], axis=1)
    fr = jnp.maximum(
        jnp.dot(fused, wfuse[...], preferred_element_type=jnp.float32)
        + bfuse[...], 0.0)
    fp = jax.nn.sigmoid(
        jnp.dot(fr, wclsf[...], preferred_element_type=jnp.float32) + bclsf[...])
    iso = deg[...] <= DEG_THRESH
    expert_ref[...] = jnp.where(iso, fr, h)
    prob_ref[...] = jnp.where(iso, fp, gp)


def _final(aggr, cntb, embb, degree, cat_repr, num_repr, des_repr, post_repr,
           dm, pm, W_self, b_graph, W_cls_g, b_cls_g,
           W_fuse, b_fuse, W_cls_f, b_cls_f):
    row = lambda i: (i, 0)
    whole = lambda i: (0, 0)
    return pl.pallas_call(
        _final_body,
        grid=(B // BM,),
        in_specs=[
            pl.BlockSpec((BM, D), row), pl.BlockSpec((BM, 1), row),
            pl.BlockSpec((BM, D), row), pl.BlockSpec((BM, 1), row),
            pl.BlockSpec((BM, D), row), pl.BlockSpec((BM, D), row),
            pl.BlockSpec((BM, D), row), pl.BlockSpec((BM, D), row),
            pl.BlockSpec((BM, 1), row), pl.BlockSpec((BM, 1), row),
            pl.BlockSpec((D, D), whole), pl.BlockSpec((1, D), whole),
            pl.BlockSpec((D, 1), whole), pl.BlockSpec((1, 1), whole),
            pl.BlockSpec((4 * D, D), whole), pl.BlockSpec((1, D), whole),
            pl.BlockSpec((D, 1), whole), pl.BlockSpec((1, 1), whole),
        ],
        out_specs=[pl.BlockSpec((BM, D), row), pl.BlockSpec((BM, 1), row)],
        out_shape=[
            jax.ShapeDtypeStruct((B, D), jnp.float32),
            jax.ShapeDtypeStruct((B, 1), jnp.float32),
        ],
    )(aggr, cntb, embb, degree, cat_repr, num_repr, des_repr, post_repr,
      dm, pm, W_self, b_graph, W_cls_g, b_cls_g,
      W_fuse, b_fuse, W_cls_f, b_cls_f)


def kernel(node_indices, degree, cat_repr, num_repr, des_repr, post_repr,
           des_mask, post_mask, edge_index, edge_type,
           W_fuse, b_fuse, W_cls_f, b_cls_f,
           node_emb, W_rel, W_self, b_graph, W_cls_g, b_cls_g):
    nidx = node_indices.astype(jnp.int32)
    srce = jnp.pad(edge_index[0].astype(jnp.int32), (0, EP - E))
    dste = jnp.pad(edge_index[1].astype(jnp.int32), (0, EP - E))
    typee = jnp.pad(edge_type.astype(jnp.int32), (0, EP - E))

    hall = _hall(node_emb, W_rel)
    slote, ride, cnts, slots = _route_pass(nidx, srce, dste, typee)
    accs = _scatter_pass(hall, slote.reshape(EP // 128, 128),
                         ride.reshape(EP // 128, 128))
    aggr, cntb, embb = _gather_pass(
        slots.reshape(128, 128), nidx.reshape(128, 128),
        accs[0], accs[1], cnts[0], cnts[1], node_emb)

    dm = des_mask.astype(jnp.float32)[:, None]
    pm = post_mask.astype(jnp.float32)[:, None]
    deg2 = degree.astype(jnp.int32)[:, None]
    return _final(aggr, cntb[:, None], embb, deg2,
                  cat_repr, num_repr, des_repr, post_repr, dm, pm,
                  W_self, b_graph.reshape(1, D), W_cls_g,
                  b_cls_g.reshape(1, 1),
                  W_fuse, b_fuse.reshape(1, D), W_cls_f,
                  b_cls_f.reshape(1, 1))


# speedup vs baseline: 18.0278x; 1.8760x over previous
import dataclasses
import functools

import jax
import jax.numpy as jnp
from jax import lax
from jax.experimental import pallas as pl
from jax.experimental.pallas import tpu as pltpu
from jax.experimental.pallas import tpu_sc as plsc

N = 50000
E = 800000
R = 4
D = 64
B = 16384
DEG_THRESH = 20

NTILES = 32
EP = 819200
EPT = EP // NTILES
CH1 = 1024
NCH1 = EPT // CH1
CCAP = 26624
BP = 16512
DUMMY = B
CR = 1152
BPT = B // NTILES
HREL = R * N
MARKN = 50048
RPT = BP // 16


def _sc_compiler_params():
    cp = pltpu.CompilerParams()
    fields = pltpu.CompilerParams.__dataclass_fields__
    if "needs_layout_passes" in fields:
        cp = dataclasses.replace(cp, needs_layout_passes=False)
    if "use_tc_tiling_on_sc" in fields:
        cp = dataclasses.replace(cp, use_tc_tiling_on_sc=False)
    return cp


def _sc_mesh():
    return plsc.VectorSubcoreMesh(core_axis_name="c", subcore_axis_name="s")


def _hall_body(emb, wrel, out):
    out[...] = jnp.dot(emb[...], wrel[0], preferred_element_type=jnp.float32)


def _hall(node_emb, W_rel):
    nb = 25
    bm = N // nb
    return pl.pallas_call(
        _hall_body,
        grid=(R, nb),
        in_specs=[pl.BlockSpec((bm, D), lambda r, i: (i, 0)),
                  pl.BlockSpec((1, D, D), lambda r, i: (r, 0, 0))],
        out_specs=pl.BlockSpec((bm, D), lambda r, i: (r * nb + i, 0)),
        out_shape=jax.ShapeDtypeStruct((HREL, D), jnp.float32),
    )(node_emb, W_rel)


def _route_pass(nidx, srce, dste, typee):
    out_type = [
        jax.ShapeDtypeStruct((EP,), jnp.int32),
        jax.ShapeDtypeStruct((EP,), jnp.int32),
        jax.ShapeDtypeStruct((NTILES, 16), jnp.int32),
        jax.ShapeDtypeStruct((2, CR, 16), jnp.float32),
        jax.ShapeDtypeStruct((B,), jnp.int32),
    ]
    scratch = [
        pltpu.VMEM((MARKN,), jnp.int32),
        pltpu.VMEM((512,), jnp.int32),
        pltpu.VMEM((CH1,), jnp.int32),
        pltpu.VMEM((CH1,), jnp.int32),
        pltpu.VMEM((CH1,), jnp.int32),
        pltpu.VMEM((CCAP,), jnp.int32),
        pltpu.VMEM((CCAP,), jnp.int32),
        pltpu.VMEM((CR, 16), jnp.float32),
        pltpu.VMEM((8, 16), jnp.float32),
        pltpu.VMEM((9, 128), jnp.int32),
        pltpu.VMEM((512,), jnp.int32),
        pltpu.VMEM((16,), jnp.int32),
        pltpu.VMEM_SHARED((CR, 16), jnp.float32),
    ]

    @functools.partial(pl.kernel, mesh=_sc_mesh(), out_type=out_type,
                       scratch_types=scratch,
                       compiler_params=_sc_compiler_params())
    def body(nidx_h, src_h, dst_h, typ_h,
             slote_out, ride_out, kcnt_out, cnt_out, slots_out,
             mark, nbuf, esrc, edst, etyp, sloto, rido, cnt_v,
             zcnt, ident, slots_v, kbuf, cnt_sh):
        cid = lax.axis_index("c")
        sid = lax.axis_index("s")
        w = cid * 16 + sid
        i16 = lax.iota(jnp.int32, 16)
        zf16 = jnp.zeros((16,), jnp.float32)
        ones16 = jnp.ones((16,), jnp.float32)

        @pl.loop(0, 8)
        def _(rr):
            zcnt[rr, pl.ds(0, 16)] = zf16

        @pl.loop(0, CR)
        def _(rr):
            cnt_v[rr, pl.ds(0, 16)] = zf16

        @pl.loop(0, MARKN, step=16)
        def _(i):
            mark[pl.ds(i, 16)] = jnp.full((16,), -1, jnp.int32)

        for j in range(9):
            @pl.loop(0, 128, step=16)
            def _(o, j=j):
                ident[j, pl.ds(o, 16)] = (j * 128 + o) + i16

        for j in range(9):
            pltpu.sync_copy(zcnt, cnt_sh.at[pl.ds(sid * 72 + j * 8, 8)])

        for bk in range(B // 512):
            pltpu.sync_copy(nidx_h.at[pl.ds(bk * 512, 512)], nbuf)

            @pl.loop(0, 512, step=16)
            def _(i, bk=bk):
                idx = nbuf[pl.ds(i, 16)]
                plsc.store_scatter(mark, [idx], bk * 512 + i + i16)

        pltpu.sync_copy(nidx_h.at[pl.ds(w * BPT, BPT)], nbuf)

        @pl.loop(0, BPT, step=16)
        def _(i):
            nv = nbuf[pl.ds(i, 16)]
            slots_v[pl.ds(i, 16)] = plsc.load_gather(mark, [nv])

        pltpu.sync_copy(slots_v, slots_out.at[pl.ds(w * BPT, BPT)])

        def chunk_body(c, cur):
            base = w * EPT + c * CH1
            pltpu.sync_copy(src_h.at[pl.ds(base, CH1)], esrc)
            pltpu.sync_copy(dst_h.at[pl.ds(base, CH1)], edst)
            pltpu.sync_copy(typ_h.at[pl.ds(base, CH1)], etyp)

            def grp(oi, cur):
                o = oi * 16
                d16 = edst[pl.ds(o, 16)]
                m = plsc.load_gather(mark, [d16])
                g = base + o + i16
                keep = (m >= 0) & (g < E)
                mm = jnp.where(keep, m, 0)
                rid = etyp[pl.ds(o, 16)] * N + esrc[pl.ds(o, 16)]
                plsc.store_compressed(sloto.at[pl.ds(cur, 16)], mm, mask=keep)
                plsc.store_compressed(rido.at[pl.ds(cur, 16)], rid, mask=keep)
                plsc.addupdate_scatter(
                    cnt_v,
                    [lax.shift_right_logical(mm, 4), lax.bitwise_and(mm, 15)],
                    ones16, mask=keep)
                return cur + jnp.sum(keep.astype(jnp.int32))

            return lax.fori_loop(0, CH1 // 16, grp, cur)

        kept = lax.fori_loop(0, NCH1, chunk_body, jnp.int32(0))

        dummy16 = jnp.full((16,), DUMMY, jnp.int32)
        zero16 = jnp.zeros((16,), jnp.int32)
        for k in range(8):
            sloto[pl.ds(kept + k * 16, 16)] = dummy16
            rido[pl.ds(kept + k * 16, 16)] = zero16

        nwr = (kept + 128 + 1023) // 1024

        @pl.loop(0, nwr)
        def _(i):
            pltpu.sync_copy(sloto.at[pl.ds(i * 1024, 1024)],
                            slote_out.at[pl.ds(w * EPT + i * 1024, 1024)])
            pltpu.sync_copy(rido.at[pl.ds(i * 1024, 1024)],
                            ride_out.at[pl.ds(w * EPT + i * 1024, 1024)])

        kbuf[pl.ds(0, 16)] = jnp.where(i16 == 0, kept, 0)
        pltpu.sync_copy(kbuf, kcnt_out.at[w])

        plsc.subcore_barrier()
        for j in range(9):
            pltpu.sync_copy(cnt_v.at[pl.ds(j * 128, 128)],
                            cnt_sh.at[ident.at[j]], add=True)
        plsc.subcore_barrier()
        pltpu.sync_copy(cnt_sh.at[pl.ds(sid * 72, 72)],
                        cnt_out.at[cid, pl.ds(sid * 72, 72)])

    return body(nidx, srce, dste, typee)


def _scatter_pass(hall, slote2d, ride2d, kcnt):
    out_type = jax.ShapeDtypeStruct((2, BP, D), jnp.float32)
    scratch = [
        pltpu.VMEM((32, 16), jnp.int32),
        pltpu.VMEM((1, 128), jnp.int32),
        pltpu.VMEM((1, 128), jnp.int32),
        pltpu.VMEM((128, D), jnp.float32),
        pltpu.VMEM_SHARED((BP, D), jnp.float32),
    ]

    @functools.partial(pl.kernel, mesh=_sc_mesh(), out_type=out_type,
                       scratch_types=scratch,
                       compiler_params=_sc_compiler_params())
    def body(hall_h, slote_h, ride_h, kcnt_h, acc_out,
             kcnt_v, slot_b, rid_b, rows_v, acc_sh):
        cid = lax.axis_index("c")
        sid = lax.axis_index("s")
        w = cid * 16 + sid
        i16 = lax.iota(jnp.int32, 16)
        zf16 = jnp.zeros((16,), jnp.float32)

        @pl.loop(0, 128)
        def _(rr):
            for cc in range(4):
                rows_v[rr, pl.ds(cc * 16, 16)] = zf16

        for off in range(0, RPT - 8, 128):
            pltpu.sync_copy(rows_v, acc_sh.at[pl.ds(sid * RPT + off, 128)])
        pltpu.sync_copy(rows_v.at[pl.ds(0, 8)],
                        acc_sh.at[pl.ds(sid * RPT + RPT - 8, 8)])

        pltpu.sync_copy(kcnt_h, kcnt_v)
        kv = kcnt_v[w, pl.ds(0, 16)]
        kept = jnp.sum(jnp.where(i16 == 0, kv, 0))
        ngrp = (kept + 127) // 128

        plsc.subcore_barrier()

        @pl.loop(0, ngrp)
        def _(g):
            rowbase = w * (EPT // 128) + g
            pltpu.sync_copy(slote_h.at[pl.ds(rowbase, 1)], slot_b)
            pltpu.sync_copy(ride_h.at[pl.ds(rowbase, 1)], rid_b)
            pltpu.sync_copy(hall_h.at[rid_b.at[0]], rows_v)
            pltpu.sync_copy(rows_v, acc_sh.at[slot_b.at[0]], add=True)

        plsc.subcore_barrier()
        pltpu.sync_copy(acc_sh.at[pl.ds(sid * RPT, RPT)],
                        acc_out.at[cid, pl.ds(sid * RPT, RPT)])

    return body(hall, slote2d, ride2d, kcnt)


def _gather_pass(slots2d, nidx2d, acc0, acc1, cnt0, cnt1, node_emb):
    out_type = [
        jax.ShapeDtypeStruct((B, D), jnp.float32),
        jax.ShapeDtypeStruct((B,), jnp.float32),
        jax.ShapeDtypeStruct((B, D), jnp.float32),
    ]
    scratch = [
        pltpu.VMEM((4, 128), jnp.int32),
        pltpu.VMEM((4, 128), jnp.int32),
        pltpu.VMEM((256, D), jnp.float32),
        pltpu.VMEM((256, D), jnp.float32),
        pltpu.VMEM((256, D), jnp.float32),
        pltpu.VMEM((CR, 16), jnp.float32),
        pltpu.VMEM((CR, 16), jnp.float32),
        pltpu.VMEM((512,), jnp.float32),
    ]

    @functools.partial(pl.kernel, mesh=_sc_mesh(), out_type=out_type,
                       scratch_types=scratch,
                       compiler_params=_sc_compiler_params())
    def body(slots_h, nidx_h, acc0_h, acc1_h, cnt0_h, cnt1_h, emb_h,
             agg_out, cntb_out, embb_out,
             slots_v, nidx_v, rows0, rows1, erows, cnt0_v, cnt1_v, cntb_v):
        cid = lax.axis_index("c")
        sid = lax.axis_index("s")
        w = cid * 16 + sid
        pltpu.sync_copy(slots_h.at[pl.ds(w * 4, 4)], slots_v)
        pltpu.sync_copy(nidx_h.at[pl.ds(w * 4, 4)], nidx_v)
        pltpu.sync_copy(cnt0_h, cnt0_v)
        pltpu.sync_copy(cnt1_h, cnt1_v)
        for k in range(2):
            for j in range(2):
                rr = k * 2 + j
                pltpu.sync_copy(acc0_h.at[slots_v.at[rr]],
                                rows0.at[pl.ds(j * 128, 128)])
                pltpu.sync_copy(acc1_h.at[slots_v.at[rr]],
                                rows1.at[pl.ds(j * 128, 128)])
                pltpu.sync_copy(emb_h.at[nidx_v.at[rr]],
                                erows.at[pl.ds(j * 128, 128)])

            @pl.loop(0, 256)
            def _(rr):
                for cc in range(4):
                    sl = pl.ds(cc * 16, 16)
                    rows0[rr, sl] = rows0[rr, sl] + rows1[rr, sl]

            for j in range(2):
                rr = k * 2 + j

                @pl.loop(0, 128, step=16)
                def _(o, rr=rr, j=j, k=k):
                    s16 = slots_v[rr, pl.ds(o, 16)]
                    hi = lax.shift_right_logical(s16, 4)
                    lo = lax.bitwise_and(s16, 15)
                    c0 = plsc.load_gather(cnt0_v, [hi, lo])
                    c1 = plsc.load_gather(cnt1_v, [hi, lo])
                    cntb_v[pl.ds(k * 256 + j * 128 + o, 16)] = c0 + c1

            pltpu.sync_copy(rows0, agg_out.at[pl.ds(w * 512 + k * 256, 256)])
            pltpu.sync_copy(erows, embb_out.at[pl.ds(w * 512 + k * 256, 256)])
        pltpu.sync_copy(cntb_v, cntb_out.at[pl.ds(w * 512, 512)])

    return body(slots2d, nidx2d, acc0, acc1, cnt0, cnt1, node_emb)


BM = 2048


def _final_body(aggr, cntb, embb, deg, cat, num, des, post, dm, pm,
                wself, bgraph, wclsg, bclsg, wfuse, bfuse, wclsf, bclsf,
                expert_ref, prob_ref):
    agg = aggr[...] / jnp.maximum(cntb[...], 1.0)
    h = jnp.maximum(
        agg + jnp.dot(embb[...], wself[...],
                      preferred_element_type=jnp.float32) + bgraph[...], 0.0)
    gp = jax.nn.sigmoid(
        jnp.dot(h, wclsg[...], preferred_element_type=jnp.float32) + bclsg[...])
    fused = jnp.concatenate(
        [cat[...], num[...], des[...] * dm[...], post[...] * pm[...]], axis=1)
    fr = jnp.maximum(
        jnp.dot(fused, wfuse[...], preferred_element_type=jnp.float32)
        + bfuse[...], 0.0)
    fp = jax.nn.sigmoid(
        jnp.dot(fr, wclsf[...], preferred_element_type=jnp.float32) + bclsf[...])
    iso = deg[...] <= DEG_THRESH
    expert_ref[...] = jnp.where(iso, fr, h)
    prob_ref[...] = jnp.where(iso, fp, gp)


def _final(aggr, cntb, embb, degree, cat_repr, num_repr, des_repr, post_repr,
           dm, pm, W_self, b_graph, W_cls_g, b_cls_g,
           W_fuse, b_fuse, W_cls_f, b_cls_f):
    row = lambda i: (i, 0)
    whole = lambda i: (0, 0)
    return pl.pallas_call(
        _final_body,
        grid=(B // BM,),
        in_specs=[
            pl.BlockSpec((BM, D), row), pl.BlockSpec((BM, 1), row),
            pl.BlockSpec((BM, D), row), pl.BlockSpec((BM, 1), row),
            pl.BlockSpec((BM, D), row), pl.BlockSpec((BM, D), row),
            pl.BlockSpec((BM, D), row), pl.BlockSpec((BM, D), row),
            pl.BlockSpec((BM, 1), row), pl.BlockSpec((BM, 1), row),
            pl.BlockSpec((D, D), whole), pl.BlockSpec((1, D), whole),
            pl.BlockSpec((D, 1), whole), pl.BlockSpec((1, 1), whole),
            pl.BlockSpec((4 * D, D), whole), pl.BlockSpec((1, D), whole),
            pl.BlockSpec((D, 1), whole), pl.BlockSpec((1, 1), whole),
        ],
        out_specs=[pl.BlockSpec((BM, D), row), pl.BlockSpec((BM, 1), row)],
        out_shape=[
            jax.ShapeDtypeStruct((B, D), jnp.float32),
            jax.ShapeDtypeStruct((B, 1), jnp.float32),
        ],
    )(aggr, cntb, embb, degree, cat_repr, num_repr, des_repr, post_repr,
      dm, pm, W_self, b_graph, W_cls_g, b_cls_g,
      W_fuse, b_fuse, W_cls_f, b_cls_f)


def kernel(node_indices, degree, cat_repr, num_repr, des_repr, post_repr,
           des_mask, post_mask, edge_index, edge_type,
           W_fuse, b_fuse, W_cls_f, b_cls_f,
           node_emb, W_rel, W_self, b_graph, W_cls_g, b_cls_g):
    nidx = node_indices.astype(jnp.int32)
    srce = jnp.pad(edge_index[0].astype(jnp.int32), (0, EP - E))
    dste = jnp.pad(edge_index[1].astype(jnp.int32), (0, EP - E))
    typee = jnp.pad(edge_type.astype(jnp.int32), (0, EP - E))

    hall = _hall(node_emb, W_rel)
    slote, ride, kcnt, cnts, slots = _route_pass(nidx, srce, dste, typee)
    accs = _scatter_pass(hall, slote.reshape(EP // 128, 128),
                         ride.reshape(EP // 128, 128), kcnt)
    aggr, cntb, embb = _gather_pass(
        slots.reshape(128, 128), nidx.reshape(128, 128),
        accs[0], accs[1], cnts[0], cnts[1], node_emb)

    dm = des_mask.astype(jnp.float32)[:, None]
    pm = post_mask.astype(jnp.float32)[:, None]
    deg2 = degree.astype(jnp.int32)[:, None]
    return _final(aggr, cntb[:, None], embb, deg2,
                  cat_repr, num_repr, des_repr, post_repr, dm, pm,
                  W_self, b_graph.reshape(1, D), W_cls_g,
                  b_cls_g.reshape(1, 1),
                  W_fuse, b_fuse.reshape(1, D), W_cls_f,
                  b_cls_f.reshape(1, 1))


# speedup vs baseline: 19.3368x; 1.0726x over previous
import dataclasses
import functools

import jax
import jax.numpy as jnp
from jax import lax
from jax.experimental import pallas as pl
from jax.experimental.pallas import tpu as pltpu
from jax.experimental.pallas import tpu_sc as plsc

N = 50000
E = 800000
R = 4
D = 64
B = 16384
DEG_THRESH = 20

NTILES = 32
EP = 819200
EPT = EP // NTILES
CH1 = 1024
NCH1 = EPT // CH1
CCAP = 26624
CH2 = 256
BP = 16512
DUMMY = B
CR = 1152
BPT = B // NTILES
HREL = R * N
MARKN = 50048
RPT = BP // 16


def _sc_compiler_params():
    cp = pltpu.CompilerParams()
    fields = pltpu.CompilerParams.__dataclass_fields__
    if "needs_layout_passes" in fields:
        cp = dataclasses.replace(cp, needs_layout_passes=False)
    if "use_tc_tiling_on_sc" in fields:
        cp = dataclasses.replace(cp, use_tc_tiling_on_sc=False)
    return cp


def _sc_mesh():
    return plsc.VectorSubcoreMesh(core_axis_name="c", subcore_axis_name="s")


def _hall_body(emb, wrel, out):
    out[...] = jnp.dot(emb[...], wrel[0], preferred_element_type=jnp.float32)


def _hall(node_emb, W_rel):
    nb = 25
    bm = N // nb
    return pl.pallas_call(
        _hall_body,
        grid=(R, nb),
        in_specs=[pl.BlockSpec((bm, D), lambda r, i: (i, 0)),
                  pl.BlockSpec((1, D, D), lambda r, i: (r, 0, 0))],
        out_specs=pl.BlockSpec((bm, D), lambda r, i: (r * nb + i, 0)),
        out_shape=jax.ShapeDtypeStruct((HREL, D), jnp.float32),
    )(node_emb, W_rel)


def _route_pass(nidx, srce, dste, typee):
    out_type = [
        jax.ShapeDtypeStruct((EP,), jnp.int32),
        jax.ShapeDtypeStruct((EP,), jnp.int32),
        jax.ShapeDtypeStruct((NTILES, 16), jnp.int32),
        jax.ShapeDtypeStruct((2, CR, 16), jnp.float32),
        jax.ShapeDtypeStruct((B,), jnp.int32),
    ]
    scratch = [
        pltpu.VMEM((MARKN,), jnp.int32),
        pltpu.VMEM((512,), jnp.int32),
        pltpu.VMEM((2, CH1), jnp.int32),
        pltpu.VMEM((2, CH1), jnp.int32),
        pltpu.VMEM((2, CH1), jnp.int32),
        pltpu.VMEM((CCAP,), jnp.int32),
        pltpu.VMEM((CCAP,), jnp.int32),
        pltpu.VMEM((CR, 16), jnp.float32),
        pltpu.VMEM((8, 16), jnp.float32),
        pltpu.VMEM((9, 128), jnp.int32),
        pltpu.VMEM((16,), jnp.int32),
        pltpu.VMEM_SHARED((CR, 16), jnp.float32),
        pltpu.SemaphoreType.DMA,
        pltpu.SemaphoreType.DMA,
        pltpu.SemaphoreType.DMA,
    ]

    @functools.partial(pl.kernel, mesh=_sc_mesh(), out_type=out_type,
                       scratch_types=scratch,
                       compiler_params=_sc_compiler_params())
    def body(nidx_h, src_h, dst_h, typ_h,
             slote_out, ride_out, kcnt_out, cnt_out, slots_out,
             mark, nbuf, esrc, edst, etyp, sloto, rido, cnt_v,
             zcnt, ident, kbuf, cnt_sh, psem0, psem1, wsem):
        cid = lax.axis_index("c")
        sid = lax.axis_index("s")
        w = cid * 16 + sid
        i16 = lax.iota(jnp.int32, 16)
        zf16 = jnp.zeros((16,), jnp.float32)
        ones16 = jnp.ones((16,), jnp.float32)

        @pl.loop(0, 8)
        def _(rr):
            zcnt[rr, pl.ds(0, 16)] = zf16

        @pl.loop(0, CR)
        def _(rr):
            cnt_v[rr, pl.ds(0, 16)] = zf16

        @pl.loop(0, MARKN, step=16)
        def _(i):
            mark[pl.ds(i, 16)] = jnp.full((16,), -1, jnp.int32)

        for j in range(9):
            @pl.loop(0, 128, step=16)
            def _(o, j=j):
                ident[j, pl.ds(o, 16)] = (j * 128 + o) + i16

        for j in range(9):
            pltpu.sync_copy(zcnt, cnt_sh.at[pl.ds(sid * 72 + j * 8, 8)])

        for bk in range(B // 512):
            pltpu.sync_copy(nidx_h.at[pl.ds(bk * 512, 512)], nbuf)

            @pl.loop(0, 512, step=16)
            def _(i, bk=bk):
                idx = nbuf[pl.ds(i, 16)]
                plsc.store_scatter(mark, [idx], bk * 512 + i + i16)

        pltpu.sync_copy(nidx_h.at[pl.ds(w * BPT, BPT)], nbuf)

        @pl.loop(0, BPT, step=16)
        def _(i):
            nv = nbuf[pl.ds(i, 16)]
            nbuf[pl.ds(i, 16)] = plsc.load_gather(mark, [nv])

        pltpu.sync_copy(nbuf, slots_out.at[pl.ds(w * BPT, BPT)])

        psems = (psem0, psem1)

        def issue_load(c):
            par = c & 1
            base = w * EPT + c * CH1
            pltpu.async_copy(src_h.at[pl.ds(base, CH1)], esrc.at[par], psems[par])
            pltpu.async_copy(dst_h.at[pl.ds(base, CH1)], edst.at[par], psems[par])
            pltpu.async_copy(typ_h.at[pl.ds(base, CH1)], etyp.at[par], psems[par])

        def wait_load(c):
            par = c & 1
            base = w * EPT + c * CH1
            pltpu.make_async_copy(src_h.at[pl.ds(base, CH1)], esrc.at[par],
                                  psems[par]).wait()
            pltpu.make_async_copy(dst_h.at[pl.ds(base, CH1)], edst.at[par],
                                  psems[par]).wait()
            pltpu.make_async_copy(typ_h.at[pl.ds(base, CH1)], etyp.at[par],
                                  psems[par]).wait()

        issue_load(0)
        cur = jnp.int32(0)
        for c in range(NCH1):
            par = c & 1
            base = w * EPT + c * CH1
            wait_load(c)
            if c + 1 < NCH1:
                issue_load(c + 1)

            def grp(oi, cur, par=par, base=base):
                o = oi * 16
                d16 = edst[par, pl.ds(o, 16)]
                m = plsc.load_gather(mark, [d16])
                g = base + o + i16
                keep = (m >= 0) & (g < E)
                mm = jnp.where(keep, m, 0)
                rid = etyp[par, pl.ds(o, 16)] * N + esrc[par, pl.ds(o, 16)]
                plsc.store_compressed(sloto.at[pl.ds(cur, 16)], mm, mask=keep)
                plsc.store_compressed(rido.at[pl.ds(cur, 16)], rid, mask=keep)
                plsc.addupdate_scatter(
                    cnt_v,
                    [lax.shift_right_logical(mm, 4), lax.bitwise_and(mm, 15)],
                    ones16, mask=keep)
                return cur + jnp.sum(keep.astype(jnp.int32))

            cur = lax.fori_loop(0, CH1 // 16, grp, cur)

        kept = cur

        dummy16 = jnp.full((16,), DUMMY, jnp.int32)
        zero16 = jnp.zeros((16,), jnp.int32)
        for k in range(CH2 // 16):
            sloto[pl.ds(kept + k * 16, 16)] = dummy16
            rido[pl.ds(kept + k * 16, 16)] = zero16

        nwr = (kept + CH2 + 1023) // 1024

        @pl.loop(0, nwr)
        def _(i):
            pltpu.async_copy(sloto.at[pl.ds(i * 1024, 1024)],
                             slote_out.at[pl.ds(w * EPT + i * 1024, 1024)],
                             wsem)
            pltpu.async_copy(rido.at[pl.ds(i * 1024, 1024)],
                             ride_out.at[pl.ds(w * EPT + i * 1024, 1024)],
                             wsem)

        @pl.loop(0, nwr)
        def _(i):
            pltpu.make_async_copy(
                sloto.at[pl.ds(i * 1024, 1024)],
                slote_out.at[pl.ds(w * EPT + i * 1024, 1024)], wsem).wait()
            pltpu.make_async_copy(
                rido.at[pl.ds(i * 1024, 1024)],
                ride_out.at[pl.ds(w * EPT + i * 1024, 1024)], wsem).wait()

        kbuf[pl.ds(0, 16)] = jnp.where(i16 == 0, kept, 0)
        pltpu.sync_copy(kbuf, kcnt_out.at[w])

        plsc.subcore_barrier()
        for j in range(9):
            pltpu.sync_copy(cnt_v.at[pl.ds(j * 128, 128)],
                            cnt_sh.at[ident.at[j]], add=True)
        plsc.subcore_barrier()
        pltpu.sync_copy(cnt_sh.at[pl.ds(sid * 72, 72)],
                        cnt_out.at[cid, pl.ds(sid * 72, 72)])

    return body(nidx, srce, dste, typee)


def _scatter_pass(hall, slote2d, ride2d, kcnt):
    out_type = jax.ShapeDtypeStruct((2, BP, D), jnp.float32)
    scratch = [
        pltpu.VMEM((32, 16), jnp.int32),
        pltpu.VMEM((2, 2, 128), jnp.int32),
        pltpu.VMEM((2, 2, 128), jnp.int32),
        pltpu.VMEM((CH2, D), jnp.float32),
        pltpu.VMEM((CH2, D), jnp.float32),
        pltpu.VMEM_SHARED((BP, D), jnp.float32),
        pltpu.SemaphoreType.DMA,
        pltpu.SemaphoreType.DMA,
        pltpu.SemaphoreType.DMA,
        pltpu.SemaphoreType.DMA,
    ]

    @functools.partial(pl.kernel, mesh=_sc_mesh(), out_type=out_type,
                       scratch_types=scratch,
                       compiler_params=_sc_compiler_params())
    def body(hall_h, slote_h, ride_h, kcnt_h, acc_out,
             kcnt_v, slot_b, rid_b, rows0, rows1, acc_sh,
             gsem0, gsem1, ssem0, ssem1):
        cid = lax.axis_index("c")
        sid = lax.axis_index("s")
        w = cid * 16 + sid
        i16 = lax.iota(jnp.int32, 16)
        zf16 = jnp.zeros((16,), jnp.float32)
        rows = (rows0, rows1)
        gsems = (gsem0, gsem1)
        ssems = (ssem0, ssem1)
        ebase = w * (EPT // 128)

        @pl.loop(0, 128)
        def _(rr):
            for cc in range(4):
                rows0[rr, pl.ds(cc * 16, 16)] = zf16

        for off in range(0, RPT - 8, 128):
            pltpu.sync_copy(rows0.at[pl.ds(0, 128)],
                            acc_sh.at[pl.ds(sid * RPT + off, 128)])
        pltpu.sync_copy(rows0.at[pl.ds(0, 8)],
                        acc_sh.at[pl.ds(sid * RPT + RPT - 8, 8)])

        pltpu.sync_copy(kcnt_h, kcnt_v)
        kv = kcnt_v[w, pl.ds(0, 16)]
        kept = jnp.sum(jnp.where(i16 == 0, kv, 0))
        nch = (kept + CH2 - 1) // CH2

        plsc.subcore_barrier()

        def idx_load(c, par):
            rb = ebase + c * 2
            pltpu.sync_copy(slote_h.at[pl.ds(rb, 2)], slot_b.at[par])
            pltpu.sync_copy(ride_h.at[pl.ds(rb, 2)], rid_b.at[par])

        def fire_gathers(par):
            for q in range(2):
                pltpu.async_copy(hall_h.at[rid_b.at[par, q]],
                                 rows[par].at[pl.ds(q * 128, 128)], gsems[par])

        def drain_gathers(par):
            for q in range(2):
                pltpu.make_async_copy(
                    hall_h.at[rid_b.at[par, q]],
                    rows[par].at[pl.ds(q * 128, 128)], gsems[par]).wait()

        def fire_scatters(par):
            for q in range(2):
                pltpu.async_copy(rows[par].at[pl.ds(q * 128, 128)],
                                 acc_sh.at[slot_b.at[par, q]], ssems[par],
                                 add=True)

        def drain_scatters(par):
            for q in range(2):
                pltpu.make_async_copy(
                    rows[par].at[pl.ds(q * 128, 128)],
                    acc_sh.at[slot_b.at[par, q]], ssems[par]).wait()

        def pair_step(p, _):
            for par in (0, 1):
                c = 2 * p + par

                @pl.when(c < nch)
                def _(c=c, par=par):
                    @pl.when(c >= 2)
                    def _():
                        drain_scatters(par)

                    idx_load(c, par)
                    fire_gathers(par)

                @pl.when((c >= 1) & (c - 1 < nch))
                def _(par=par):
                    drain_gathers(par ^ 1)
                    fire_scatters(par ^ 1)

            return 0

        lax.fori_loop(0, (nch + 2) // 2, pair_step, 0)

        for par in (0, 1):
            c1 = nch - 1
            c2 = nch - 2

            @pl.when(((c1 >= 0) & (lax.rem(c1, 2) == par))
                     | ((c2 >= 0) & (lax.rem(c2, 2) == par)))
            def _(par=par):
                drain_scatters(par)

        plsc.subcore_barrier()
        pltpu.sync_copy(acc_sh.at[pl.ds(sid * RPT, RPT)],
                        acc_out.at[cid, pl.ds(sid * RPT, RPT)])

    return body(hall, slote2d, ride2d, kcnt)


def _gather_pass(slots2d, nidx2d, acc0, acc1, cnt0, cnt1, node_emb):
    out_type = [
        jax.ShapeDtypeStruct((B, D), jnp.float32),
        jax.ShapeDtypeStruct((B,), jnp.float32),
        jax.ShapeDtypeStruct((B, D), jnp.float32),
    ]
    scratch = [
        pltpu.VMEM((4, 128), jnp.int32),
        pltpu.VMEM((4, 128), jnp.int32),
        pltpu.VMEM((256, D), jnp.float32),
        pltpu.VMEM((256, D), jnp.float32),
        pltpu.VMEM((256, D), jnp.float32),
        pltpu.VMEM((CR, 16), jnp.float32),
        pltpu.VMEM((CR, 16), jnp.float32),
        pltpu.VMEM((512,), jnp.float32),
    ]

    @functools.partial(pl.kernel, mesh=_sc_mesh(), out_type=out_type,
                       scratch_types=scratch,
                       compiler_params=_sc_compiler_params())
    def body(slots_h, nidx_h, acc0_h, acc1_h, cnt0_h, cnt1_h, emb_h,
             agg_out, cntb_out, embb_out,
             slots_v, nidx_v, rows0, rows1, erows, cnt0_v, cnt1_v, cntb_v):
        cid = lax.axis_index("c")
        sid = lax.axis_index("s")
        w = cid * 16 + sid
        pltpu.sync_copy(slots_h.at[pl.ds(w * 4, 4)], slots_v)
        pltpu.sync_copy(nidx_h.at[pl.ds(w * 4, 4)], nidx_v)
        pltpu.sync_copy(cnt0_h, cnt0_v)
        pltpu.sync_copy(cnt1_h, cnt1_v)
        for k in range(2):
            for j in range(2):
                rr = k * 2 + j
                pltpu.sync_copy(acc0_h.at[slots_v.at[rr]],
                                rows0.at[pl.ds(j * 128, 128)])
                pltpu.sync_copy(acc1_h.at[slots_v.at[rr]],
                                rows1.at[pl.ds(j * 128, 128)])
                pltpu.sync_copy(emb_h.at[nidx_v.at[rr]],
                                erows.at[pl.ds(j * 128, 128)])

            @pl.loop(0, 256)
            def _(rr):
                for cc in range(4):
                    sl = pl.ds(cc * 16, 16)
                    rows0[rr, sl] = rows0[rr, sl] + rows1[rr, sl]

            for j in range(2):
                rr = k * 2 + j

                @pl.loop(0, 128, step=16)
                def _(o, rr=rr, j=j, k=k):
                    s16 = slots_v[rr, pl.ds(o, 16)]
                    hi = lax.shift_right_logical(s16, 4)
                    lo = lax.bitwise_and(s16, 15)
                    c0 = plsc.load_gather(cnt0_v, [hi, lo])
                    c1 = plsc.load_gather(cnt1_v, [hi, lo])
                    cntb_v[pl.ds(k * 256 + j * 128 + o, 16)] = c0 + c1

            pltpu.sync_copy(rows0, agg_out.at[pl.ds(w * 512 + k * 256, 256)])
            pltpu.sync_copy(erows, embb_out.at[pl.ds(w * 512 + k * 256, 256)])
        pltpu.sync_copy(cntb_v, cntb_out.at[pl.ds(w * 512, 512)])

    return body(slots2d, nidx2d, acc0, acc1, cnt0, cnt1, node_emb)


BM = 2048


def _final_body(aggr, cntb, embb, deg, cat, num, des, post, dm, pm,
                wself, bgraph, wclsg, bclsg, wfuse, bfuse, wclsf, bclsf,
                expert_ref, prob_ref):
    agg = aggr[...] / jnp.maximum(cntb[...], 1.0)
    h = jnp.maximum(
        agg + jnp.dot(embb[...], wself[...],
                      preferred_element_type=jnp.float32) + bgraph[...], 0.0)
    gp = jax.nn.sigmoid(
        jnp.dot(h, wclsg[...], preferred_element_type=jnp.float32) + bclsg[...])
    fused = jnp.concatenate(
        [cat[...], num[...], des[...] * dm[...], post[...] * pm[...]], axis=1)
    fr = jnp.maximum(
        jnp.dot(fused, wfuse[...], preferred_element_type=jnp.float32)
        + bfuse[...], 0.0)
    fp = jax.nn.sigmoid(
        jnp.dot(fr, wclsf[...], preferred_element_type=jnp.float32) + bclsf[...])
    iso = deg[...] <= DEG_THRESH
    expert_ref[...] = jnp.where(iso, fr, h)
    prob_ref[...] = jnp.where(iso, fp, gp)


def _final(aggr, cntb, embb, degree, cat_repr, num_repr, des_repr, post_repr,
           dm, pm, W_self, b_graph, W_cls_g, b_cls_g,
           W_fuse, b_fuse, W_cls_f, b_cls_f):
    row = lambda i: (i, 0)
    whole = lambda i: (0, 0)
    return pl.pallas_call(
        _final_body,
        grid=(B // BM,),
        in_specs=[
            pl.BlockSpec((BM, D), row), pl.BlockSpec((BM, 1), row),
            pl.BlockSpec((BM, D), row), pl.BlockSpec((BM, 1), row),
            pl.BlockSpec((BM, D), row), pl.BlockSpec((BM, D), row),
            pl.BlockSpec((BM, D), row), pl.BlockSpec((BM, D), row),
            pl.BlockSpec((BM, 1), row), pl.BlockSpec((BM, 1), row),
            pl.BlockSpec((D, D), whole), pl.BlockSpec((1, D), whole),
            pl.BlockSpec((D, 1), whole), pl.BlockSpec((1, 1), whole),
            pl.BlockSpec((4 * D, D), whole), pl.BlockSpec((1, D), whole),
            pl.BlockSpec((D, 1), whole), pl.BlockSpec((1, 1), whole),
        ],
        out_specs=[pl.BlockSpec((BM, D), row), pl.BlockSpec((BM, 1), row)],
        out_shape=[
            jax.ShapeDtypeStruct((B, D), jnp.float32),
            jax.ShapeDtypeStruct((B, 1), jnp.float32),
        ],
    )(aggr, cntb, embb, degree, cat_repr, num_repr, des_repr, post_repr,
      dm, pm, W_self, b_graph, W_cls_g, b_cls_g,
      W_fuse, b_fuse, W_cls_f, b_cls_f)


def kernel(node_indices, degree, cat_repr, num_repr, des_repr, post_repr,
           des_mask, post_mask, edge_index, edge_type,
           W_fuse, b_fuse, W_cls_f, b_cls_f,
           node_emb, W_rel, W_self, b_graph, W_cls_g, b_cls_g):
    nidx = node_indices.astype(jnp.int32)
    srce = jnp.pad(edge_index[0].astype(jnp.int32), (0, EP - E))
    dste = jnp.pad(edge_index[1].astype(jnp.int32), (0, EP - E))
    typee = jnp.pad(edge_type.astype(jnp.int32), (0, EP - E))

    hall = _hall(node_emb, W_rel)
    slote, ride, kcnt, cnts, slots = _route_pass(nidx, srce, dste, typee)
    accs = _scatter_pass(hall, slote.reshape(EP // 128, 128),
                         ride.reshape(EP // 128, 128), kcnt)
    aggr, cntb, embb = _gather_pass(
        slots.reshape(128, 128), nidx.reshape(128, 128),
        accs[0], accs[1], cnts[0], cnts[1], node_emb)

    dm = des_mask.astype(jnp.float32)[:, None]
    pm = post_mask.astype(jnp.float32)[:, None]
    deg2 = degree.astype(jnp.int32)[:, None]
    return _final(aggr, cntb[:, None], embb, deg2,
                  cat_repr, num_repr, des_repr, post_repr, dm, pm,
                  W_self, b_graph.reshape(1, D), W_cls_g,
                  b_cls_g.reshape(1, 1),
                  W_fuse, b_fuse.reshape(1, D), W_cls_f,
                  b_cls_f.reshape(1, 1))


# speedup vs baseline: 26.4618x; 1.3685x over previous
import dataclasses
import functools

import jax
import jax.numpy as jnp
from jax import lax
from jax.experimental import pallas as pl
from jax.experimental.pallas import tpu as pltpu
from jax.experimental.pallas import tpu_sc as plsc

N = 50000
E = 800000
R = 4
D = 64
B = 16384
DEG_THRESH = 20

NTILES = 32
EP = 819200
EPT = EP // NTILES
CH1 = 1024
NCH1 = EPT // CH1
CCAP = 26624
CH2 = 256
BP = 16512
DUMMY = B
CR = 1152
BPT = B // NTILES
HREL = R * N
MARKN = 50048
RPT = BP // 16


def _sc_compiler_params():
    cp = pltpu.CompilerParams()
    fields = pltpu.CompilerParams.__dataclass_fields__
    if "needs_layout_passes" in fields:
        cp = dataclasses.replace(cp, needs_layout_passes=False)
    if "use_tc_tiling_on_sc" in fields:
        cp = dataclasses.replace(cp, use_tc_tiling_on_sc=False)
    return cp


def _sc_mesh():
    return plsc.VectorSubcoreMesh(core_axis_name="c", subcore_axis_name="s")


def _hall_body(emb2, w2, out):
    out[...] = jnp.dot(emb2[...], w2[0], preferred_element_type=jnp.float32)


def _hall(node_emb, W_rel):
    emb2 = node_emb.reshape(N // 2, 2 * D)
    w2 = jnp.zeros((R, 2 * D, 2 * D), jnp.float32)
    w2 = w2.at[:, :D, :D].set(W_rel).at[:, D:, D:].set(W_rel)
    nb = 25
    bm = (N // 2) // nb
    return pl.pallas_call(
        _hall_body,
        grid=(R, nb),
        in_specs=[pl.BlockSpec((bm, 2 * D), lambda r, i: (i, 0)),
                  pl.BlockSpec((1, 2 * D, 2 * D), lambda r, i: (r, 0, 0))],
        out_specs=pl.BlockSpec((bm, 2 * D), lambda r, i: (r * nb + i, 0)),
        out_shape=jax.ShapeDtypeStruct((HREL // 2, 2 * D), jnp.float32),
    )(emb2, w2)


def _route_pass(nidx, srce, dste, typee):
    out_type = [
        jax.ShapeDtypeStruct((EP,), jnp.int32),
        jax.ShapeDtypeStruct((EP,), jnp.int32),
        jax.ShapeDtypeStruct((NTILES, 16), jnp.int32),
        jax.ShapeDtypeStruct((2, CR, 16), jnp.float32),
        jax.ShapeDtypeStruct((B,), jnp.int32),
    ]
    scratch = [
        pltpu.VMEM((MARKN,), jnp.int32),
        pltpu.VMEM((512,), jnp.int32),
        pltpu.VMEM((2, CH1), jnp.int32),
        pltpu.VMEM((2, CH1), jnp.int32),
        pltpu.VMEM((2, CH1), jnp.int32),
        pltpu.VMEM((CCAP,), jnp.int32),
        pltpu.VMEM((CCAP,), jnp.int32),
        pltpu.VMEM((CR, 16), jnp.float32),
        pltpu.VMEM((8, 16), jnp.float32),
        pltpu.VMEM((9, 128), jnp.int32),
        pltpu.VMEM((16,), jnp.int32),
        pltpu.VMEM_SHARED((CR, 16), jnp.float32),
        pltpu.SemaphoreType.DMA,
        pltpu.SemaphoreType.DMA,
        pltpu.SemaphoreType.DMA,
    ]

    @functools.partial(pl.kernel, mesh=_sc_mesh(), out_type=out_type,
                       scratch_types=scratch,
                       compiler_params=_sc_compiler_params())
    def body(nidx_h, src_h, dst_h, typ_h,
             slote_out, ride_out, kcnt_out, cnt_out, slots_out,
             mark, nbuf, esrc, edst, etyp, sloto, rido, cnt_v,
             zcnt, ident, kbuf, cnt_sh, psem0, psem1, wsem):
        cid = lax.axis_index("c")
        sid = lax.axis_index("s")
        w = cid * 16 + sid
        i16 = lax.iota(jnp.int32, 16)
        zf16 = jnp.zeros((16,), jnp.float32)
        ones16 = jnp.ones((16,), jnp.float32)

        @pl.loop(0, 8)
        def _(rr):
            zcnt[rr, pl.ds(0, 16)] = zf16

        @pl.loop(0, CR)
        def _(rr):
            cnt_v[rr, pl.ds(0, 16)] = zf16

        @pl.loop(0, MARKN, step=16)
        def _(i):
            mark[pl.ds(i, 16)] = jnp.full((16,), -1, jnp.int32)

        for j in range(9):
            @pl.loop(0, 128, step=16)
            def _(o, j=j):
                ident[j, pl.ds(o, 16)] = (j * 128 + o) + i16

        for j in range(9):
            pltpu.sync_copy(zcnt, cnt_sh.at[pl.ds(sid * 72 + j * 8, 8)])

        for bk in range(B // 512):
            pltpu.sync_copy(nidx_h.at[pl.ds(bk * 512, 512)], nbuf)

            @pl.loop(0, 512, step=16)
            def _(i, bk=bk):
                idx = nbuf[pl.ds(i, 16)]
                plsc.store_scatter(mark, [idx], bk * 512 + i + i16)

        pltpu.sync_copy(nidx_h.at[pl.ds(w * BPT, BPT)], nbuf)

        @pl.loop(0, BPT, step=16)
        def _(i):
            nv = nbuf[pl.ds(i, 16)]
            nbuf[pl.ds(i, 16)] = plsc.load_gather(mark, [nv])

        pltpu.sync_copy(nbuf, slots_out.at[pl.ds(w * BPT, BPT)])

        psems = (psem0, psem1)

        def issue_load(c):
            par = c & 1
            base = w * EPT + c * CH1
            pltpu.async_copy(src_h.at[pl.ds(base, CH1)], esrc.at[par], psems[par])
            pltpu.async_copy(dst_h.at[pl.ds(base, CH1)], edst.at[par], psems[par])
            pltpu.async_copy(typ_h.at[pl.ds(base, CH1)], etyp.at[par], psems[par])

        def wait_load(c):
            par = c & 1
            base = w * EPT + c * CH1
            pltpu.make_async_copy(src_h.at[pl.ds(base, CH1)], esrc.at[par],
                                  psems[par]).wait()
            pltpu.make_async_copy(dst_h.at[pl.ds(base, CH1)], edst.at[par],
                                  psems[par]).wait()
            pltpu.make_async_copy(typ_h.at[pl.ds(base, CH1)], etyp.at[par],
                                  psems[par]).wait()

        issue_load(0)
        cur = jnp.int32(0)
        for c in range(NCH1):
            par = c & 1
            base = w * EPT + c * CH1
            wait_load(c)
            if c + 1 < NCH1:
                issue_load(c + 1)

            def grp(oi, cur, par=par, base=base):
                o = oi * 16
                d16 = edst[par, pl.ds(o, 16)]
                m = plsc.load_gather(mark, [d16])
                g = base + o + i16
                keep = (m >= 0) & (g < E)
                mm = jnp.where(keep, m, 0)
                rid = etyp[par, pl.ds(o, 16)] * N + esrc[par, pl.ds(o, 16)]
                plsc.store_compressed(sloto.at[pl.ds(cur, 16)], mm, mask=keep)
                plsc.store_compressed(rido.at[pl.ds(cur, 16)], rid, mask=keep)
                plsc.addupdate_scatter(
                    cnt_v,
                    [lax.shift_right_logical(mm, 4), lax.bitwise_and(mm, 15)],
                    ones16, mask=keep)
                return cur + jnp.sum(keep.astype(jnp.int32))

            cur = lax.fori_loop(0, CH1 // 16, grp, cur)

        kept = cur

        dummy16 = jnp.full((16,), DUMMY, jnp.int32)
        zero16 = jnp.zeros((16,), jnp.int32)
        for k in range(CH2 // 16):
            sloto[pl.ds(kept + k * 16, 16)] = dummy16
            rido[pl.ds(kept + k * 16, 16)] = zero16

        nwr = (kept + CH2 + 1023) // 1024

        @pl.loop(0, nwr)
        def _(i):
            pltpu.async_copy(sloto.at[pl.ds(i * 1024, 1024)],
                             slote_out.at[pl.ds(w * EPT + i * 1024, 1024)],
                             wsem)
            pltpu.async_copy(rido.at[pl.ds(i * 1024, 1024)],
                             ride_out.at[pl.ds(w * EPT + i * 1024, 1024)],
                             wsem)

        @pl.loop(0, nwr)
        def _(i):
            pltpu.make_async_copy(
                sloto.at[pl.ds(i * 1024, 1024)],
                slote_out.at[pl.ds(w * EPT + i * 1024, 1024)], wsem).wait()
            pltpu.make_async_copy(
                rido.at[pl.ds(i * 1024, 1024)],
                ride_out.at[pl.ds(w * EPT + i * 1024, 1024)], wsem).wait()

        kbuf[pl.ds(0, 16)] = jnp.where(i16 == 0, kept, 0)
        pltpu.sync_copy(kbuf, kcnt_out.at[w])

        plsc.subcore_barrier()
        for j in range(9):
            pltpu.sync_copy(cnt_v.at[pl.ds(j * 128, 128)],
                            cnt_sh.at[ident.at[j]], add=True)
        plsc.subcore_barrier()
        pltpu.sync_copy(cnt_sh.at[pl.ds(sid * 72, 72)],
                        cnt_out.at[cid, pl.ds(sid * 72, 72)])

    return body(nidx, srce, dste, typee)


def _scatter_pass(hall, slote2d, ride2d, kcnt):
    out_type = jax.ShapeDtypeStruct((2, BP, D), jnp.float32)
    scratch = [
        pltpu.VMEM((32, 16), jnp.int32),
        pltpu.VMEM((16, 128), jnp.int32),
        pltpu.VMEM((16, 128), jnp.int32),
        pltpu.VMEM((CH2, D), jnp.float32),
        pltpu.VMEM((CH2, D), jnp.float32),
        pltpu.VMEM_SHARED((BP, D), jnp.float32),
        pltpu.SemaphoreType.DMA,
        pltpu.SemaphoreType.DMA,
        pltpu.SemaphoreType.DMA,
        pltpu.SemaphoreType.DMA,
    ]

    @functools.partial(pl.kernel, mesh=_sc_mesh(), out_type=out_type,
                       scratch_types=scratch,
                       compiler_params=_sc_compiler_params())
    def body(hall_h, slote_h, ride_h, kcnt_h, acc_out,
             kcnt_v, slot_b, rid_b, rows0, rows1, acc_sh,
             gsem0, gsem1, ssem0, ssem1):
        cid = lax.axis_index("c")
        sid = lax.axis_index("s")
        w = cid * 16 + sid
        i16 = lax.iota(jnp.int32, 16)
        zf16 = jnp.zeros((16,), jnp.float32)
        rows = (rows0, rows1)
        gsems = (gsem0, gsem1)
        ssems = (ssem0, ssem1)
        ebase = w * (EPT // 128)

        @pl.loop(0, 128)
        def _(rr):
            for cc in range(4):
                rows0[rr, pl.ds(cc * 16, 16)] = zf16

        for off in range(0, RPT - 8, 128):
            pltpu.sync_copy(rows0.at[pl.ds(0, 128)],
                            acc_sh.at[pl.ds(sid * RPT + off, 128)])
        pltpu.sync_copy(rows0.at[pl.ds(0, 8)],
                        acc_sh.at[pl.ds(sid * RPT + RPT - 8, 8)])

        pltpu.sync_copy(kcnt_h, kcnt_v)
        kv = kcnt_v[w, pl.ds(0, 16)]
        kept = jnp.sum(jnp.where(i16 == 0, kv, 0))
        nch = (kept + CH2 - 1) // CH2
        nsup = (nch + 7) // 8

        plsc.subcore_barrier()

        def fire_gathers(k, par):
            for q in range(2):
                pltpu.async_copy(hall_h.at[rid_b.at[2 * k + q]],
                                 rows[par].at[pl.ds(q * 128, 128)], gsems[par])

        def drain_gathers(k, par):
            for q in range(2):
                pltpu.make_async_copy(
                    hall_h.at[rid_b.at[2 * k + q]],
                    rows[par].at[pl.ds(q * 128, 128)], gsems[par]).wait()

        def fire_scatters(k, par):
            for q in range(2):
                pltpu.async_copy(rows[par].at[pl.ds(q * 128, 128)],
                                 acc_sh.at[slot_b.at[2 * k + q]], ssems[par],
                                 add=True)

        def drain_scatters(k, par):
            for q in range(2):
                pltpu.make_async_copy(
                    rows[par].at[pl.ds(q * 128, 128)],
                    acc_sh.at[slot_b.at[2 * k + q]], ssems[par]).wait()

        def sup_step(s, _):
            rb = ebase + s * 16
            pltpu.sync_copy(slote_h.at[pl.ds(rb, 16)], slot_b)
            pltpu.sync_copy(ride_h.at[pl.ds(rb, 16)], rid_b)
            kc = jnp.minimum(nch - s * 8, 8)
            for k in range(8):
                par = k & 1

                @pl.when(k < kc)
                def _(k=k, par=par):
                    if k >= 2:
                        drain_scatters(k - 2, par)
                    fire_gathers(k, par)

                if k >= 1:
                    @pl.when(k < kc)
                    def _(k=k):
                        drain_gathers(k - 1, (k - 1) & 1)
                        fire_scatters(k - 1, (k - 1) & 1)

            for par in (0, 1):
                @pl.when((kc >= 1) & (lax.rem(kc - 1, 2) == par))
                def _(par=par):
                    drain_gathers(kc - 1, par)
                    fire_scatters(kc - 1, par)

            for par in (0, 1):
                @pl.when(kc > par)
                def _(par=par):
                    drain_scatters(kc - 1 - lax.rem(kc - 1 - par, 2), par)

            return 0

        lax.fori_loop(0, nsup, sup_step, 0)

        plsc.subcore_barrier()
        pltpu.sync_copy(acc_sh.at[pl.ds(sid * RPT, RPT)],
                        acc_out.at[cid, pl.ds(sid * RPT, RPT)])

    return body(hall, slote2d, ride2d, kcnt)


def _gather_pass(slots2d, nidx2d, accs, cnts, node_emb):
    out_type = [
        jax.ShapeDtypeStruct((B, D), jnp.float32),
        jax.ShapeDtypeStruct((B,), jnp.float32),
        jax.ShapeDtypeStruct((B, D), jnp.float32),
    ]
    scratch = [
        pltpu.VMEM((4, 128), jnp.int32),
        pltpu.VMEM((4, 128), jnp.int32),
        pltpu.VMEM((256, D), jnp.float32),
        pltpu.VMEM((256, D), jnp.float32),
        pltpu.VMEM((256, D), jnp.float32),
        pltpu.VMEM((CR, 16), jnp.float32),
        pltpu.VMEM((CR, 16), jnp.float32),
        pltpu.VMEM((512,), jnp.float32),
    ]

    @functools.partial(pl.kernel, mesh=_sc_mesh(), out_type=out_type,
                       scratch_types=scratch,
                       compiler_params=_sc_compiler_params())
    def body(slots_h, nidx_h, accs_h, cnts_h, emb_h,
             agg_out, cntb_out, embb_out,
             slots_v, nidx_v, rows0, rows1, erows, cnt0_v, cnt1_v, cntb_v):
        cid = lax.axis_index("c")
        sid = lax.axis_index("s")
        w = cid * 16 + sid
        pltpu.sync_copy(slots_h.at[pl.ds(w * 4, 4)], slots_v)
        pltpu.sync_copy(nidx_h.at[pl.ds(w * 4, 4)], nidx_v)
        pltpu.sync_copy(cnts_h.at[0], cnt0_v)
        pltpu.sync_copy(cnts_h.at[1], cnt1_v)
        for k in range(2):
            for j in range(2):
                rr = k * 2 + j
                pltpu.sync_copy(accs_h.at[0].at[slots_v.at[rr]],
                                rows0.at[pl.ds(j * 128, 128)])
                pltpu.sync_copy(accs_h.at[1].at[slots_v.at[rr]],
                                rows1.at[pl.ds(j * 128, 128)])
                pltpu.sync_copy(emb_h.at[nidx_v.at[rr]],
                                erows.at[pl.ds(j * 128, 128)])

            @pl.loop(0, 256)
            def _(rr):
                for cc in range(4):
                    sl = pl.ds(cc * 16, 16)
                    rows0[rr, sl] = rows0[rr, sl] + rows1[rr, sl]

            for j in range(2):
                rr = k * 2 + j

                @pl.loop(0, 128, step=16)
                def _(o, rr=rr, j=j, k=k):
                    s16 = slots_v[rr, pl.ds(o, 16)]
                    hi = lax.shift_right_logical(s16, 4)
                    lo = lax.bitwise_and(s16, 15)
                    c0 = plsc.load_gather(cnt0_v, [hi, lo])
                    c1 = plsc.load_gather(cnt1_v, [hi, lo])
                    cntb_v[pl.ds(k * 256 + j * 128 + o, 16)] = c0 + c1

            pltpu.sync_copy(rows0, agg_out.at[pl.ds(w * 512 + k * 256, 256)])
            pltpu.sync_copy(erows, embb_out.at[pl.ds(w * 512 + k * 256, 256)])
        pltpu.sync_copy(cntb_v, cntb_out.at[pl.ds(w * 512, 512)])

    return body(slots2d, nidx2d, accs, cnts, node_emb)


BM = 2048


def _final_body(aggr, cntb, embb, deg, cat, num, des, post, dm, pm,
                wself, bgraph, wclsg, bclsg, wfuse, bfuse, wclsf, bclsf,
                expert_ref, prob_ref):
    agg = aggr[...] / jnp.maximum(cntb[...], 1.0)
    h = jnp.maximum(
        agg + jnp.dot(embb[...], wself[...],
                      preferred_element_type=jnp.float32) + bgraph[...], 0.0)
    gp = jax.nn.sigmoid(
        jnp.dot(h, wclsg[...], preferred_element_type=jnp.float32) + bclsg[...])
    fused = jnp.concatenate(
        [cat[...], num[...], des[...] * dm[...], post[...] * pm[...]], axis=1)
    fr = jnp.maximum(
        jnp.dot(fused, wfuse[...], preferred_element_type=jnp.float32)
        + bfuse[...], 0.0)
    fp = jax.nn.sigmoid(
        jnp.dot(fr, wclsf[...], preferred_element_type=jnp.float32) + bclsf[...])
    iso = deg[...] <= DEG_THRESH
    expert_ref[...] = jnp.where(iso, fr, h)
    prob_ref[...] = jnp.where(iso, fp, gp)


def _final(aggr, cntb, embb, degree, cat_repr, num_repr, des_repr, post_repr,
           dm, pm, W_self, b_graph, W_cls_g, b_cls_g,
           W_fuse, b_fuse, W_cls_f, b_cls_f):
    row = lambda i: (i, 0)
    whole = lambda i: (0, 0)
    return pl.pallas_call(
        _final_body,
        grid=(B // BM,),
        in_specs=[
            pl.BlockSpec((BM, D), row), pl.BlockSpec((BM, 1), row),
            pl.BlockSpec((BM, D), row), pl.BlockSpec((BM, 1), row),
            pl.BlockSpec((BM, D), row), pl.BlockSpec((BM, D), row),
            pl.BlockSpec((BM, D), row), pl.BlockSpec((BM, D), row),
            pl.BlockSpec((BM, 1), row), pl.BlockSpec((BM, 1), row),
            pl.BlockSpec((D, D), whole), pl.BlockSpec((1, D), whole),
            pl.BlockSpec((D, 1), whole), pl.BlockSpec((1, 1), whole),
            pl.BlockSpec((4 * D, D), whole), pl.BlockSpec((1, D), whole),
            pl.BlockSpec((D, 1), whole), pl.BlockSpec((1, 1), whole),
        ],
        out_specs=[pl.BlockSpec((BM, D), row), pl.BlockSpec((BM, 1), row)],
        out_shape=[
            jax.ShapeDtypeStruct((B, D), jnp.float32),
            jax.ShapeDtypeStruct((B, 1), jnp.float32),
        ],
    )(aggr, cntb, embb, degree, cat_repr, num_repr, des_repr, post_repr,
      dm, pm, W_self, b_graph, W_cls_g, b_cls_g,
      W_fuse, b_fuse, W_cls_f, b_cls_f)


def kernel(node_indices, degree, cat_repr, num_repr, des_repr, post_repr,
           des_mask, post_mask, edge_index, edge_type,
           W_fuse, b_fuse, W_cls_f, b_cls_f,
           node_emb, W_rel, W_self, b_graph, W_cls_g, b_cls_g):
    nidx = node_indices.astype(jnp.int32)
    srce = jnp.pad(edge_index[0].astype(jnp.int32), (0, EP - E))
    dste = jnp.pad(edge_index[1].astype(jnp.int32), (0, EP - E))
    typee = jnp.pad(edge_type.astype(jnp.int32), (0, EP - E))

    hall2 = _hall(node_emb, W_rel)
    slote, ride, kcnt, cnts, slots = _route_pass(nidx, srce, dste, typee)
    accs = _scatter_pass(hall2.reshape(HREL, D), slote.reshape(EP // 128, 128),
                         ride.reshape(EP // 128, 128), kcnt)
    aggr, cntb, embb = _gather_pass(
        slots.reshape(128, 128), nidx.reshape(128, 128),
        accs, cnts, node_emb)

    dm = des_mask.astype(jnp.float32)[:, None]
    pm = post_mask.astype(jnp.float32)[:, None]
    deg2 = degree.astype(jnp.int32)[:, None]
    return _final(aggr, cntb[:, None], embb, deg2,
                  cat_repr, num_repr, des_repr, post_repr, dm, pm,
                  W_self, b_graph.reshape(1, D), W_cls_g,
                  b_cls_g.reshape(1, 1),
                  W_fuse, b_fuse.reshape(1, D), W_cls_f,
                  b_cls_f.reshape(1, 1))
